# Initial kernel scaffold; baseline (speedup 1.0000x reference)
import jax
import jax.numpy as jnp
from jax.experimental import pallas as pl


def kernel(dorsal, ventral, gate_w, w1, b1, w2, b2, wo, bo, wfd, wfv):
    raise NotImplementedError("write your pallas kernel here")


# speedup vs baseline: 4.5266x; 4.5266x over previous
import functools

import jax
import jax.numpy as jnp
from jax.experimental import pallas as pl
from jax.experimental.pallas import tpu as pltpu

_B = 32768
_D_DOR = 128
_D_VEN = 128
_N_EXP = 8
_D_EXP = 64
_D_OUT = 64
_FB = 0.5


def _moe_kernel(d_ref, v_ref, gwd_ref, gwv_ref, w1d_ref, w1v_ref, b1_ref,
                w2s_ref, b2_ref, woT_ref, bo_ref, wfdT_ref, wfvT_ref, exp_ref,
                assoc_ref, fbd_ref, fbv_ref, gw_ref):
    d = d_ref[...]
    v = v_ref[...]
    f32 = jnp.float32

    logits = (jnp.dot(d, gwd_ref[...], preferred_element_type=f32)
              + jnp.dot(v, gwv_ref[...], preferred_element_type=f32))

    iota = jax.lax.broadcasted_iota(jnp.int32, logits.shape, 1)
    m1 = jnp.max(logits, axis=-1, keepdims=True)
    eq1 = logits == m1
    i1 = jnp.min(jnp.where(eq1, iota, _N_EXP), axis=-1, keepdims=True)
    one1 = iota == i1
    l2 = jnp.where(one1, -jnp.inf, logits)
    m2 = jnp.max(l2, axis=-1, keepdims=True)
    eq2 = l2 == m2
    i2 = jnp.min(jnp.where(eq2, iota, _N_EXP), axis=-1, keepdims=True)
    keep = one1 | (iota == i2)

    e = jnp.where(keep, jnp.exp(logits - m1), 0.0)
    gw = e / jnp.sum(e, axis=-1, keepdims=True)
    gw_ref[...] = gw

    h = (jnp.dot(d, w1d_ref[...], preferred_element_type=f32)
         + jnp.dot(v, w1v_ref[...], preferred_element_type=f32)
         + b1_ref[...])
    h = 0.5 * h * (1.0 + jax.lax.erf(h * jnp.float32(0.7071067811865476)))

    gwx = jnp.dot(gw, exp_ref[...], preferred_element_type=f32)
    bound = (jnp.dot(h * gwx, w2s_ref[...], preferred_element_type=f32)
             + jnp.dot(gw, b2_ref[...], preferred_element_type=f32))

    assoc = jnp.dot(bound, woT_ref[...], preferred_element_type=f32) + bo_ref[...]
    assoc_ref[...] = assoc
    fbd_ref[...] = _FB * jnp.dot(assoc, wfdT_ref[...], preferred_element_type=f32)
    fbv_ref[...] = _FB * jnp.dot(assoc, wfvT_ref[...], preferred_element_type=f32)


@functools.partial(jax.jit, static_argnames=("block",))
def _run(dorsal, ventral, gate_w, w1, b1, w2, b2, wo, bo, wfd, wfv, block=1024):
    gwT = gate_w.T
    gwd, gwv = gwT[:_D_DOR], gwT[_D_DOR:]
    w1cat = w1.transpose(2, 0, 1).reshape(_D_DOR + _D_VEN, _N_EXP * _D_EXP)
    w1d, w1v = w1cat[:_D_DOR], w1cat[_D_DOR:]
    b1row = b1.reshape(1, _N_EXP * _D_EXP)
    w2s = w2.transpose(0, 2, 1).reshape(_N_EXP * _D_EXP, _D_EXP)
    woT = wo.T
    borow = bo.reshape(1, _D_OUT)
    wfdT = wfd.T
    wfvT = wfv.T
    expand = jnp.repeat(jnp.eye(_N_EXP, dtype=jnp.float32), _D_EXP, axis=1)

    grid = (_B // block,)
    tok = lambda i: (i, 0)
    full = lambda i: (0, 0)
    out_shapes = (
        jax.ShapeDtypeStruct((_B, _D_OUT), jnp.float32),
        jax.ShapeDtypeStruct((_B, _D_DOR), jnp.float32),
        jax.ShapeDtypeStruct((_B, _D_VEN), jnp.float32),
        jax.ShapeDtypeStruct((_B, _N_EXP), jnp.float32),
    )
    return pl.pallas_call(
        _moe_kernel,
        grid=grid,
        in_specs=[
            pl.BlockSpec((block, _D_DOR), tok),
            pl.BlockSpec((block, _D_VEN), tok),
            pl.BlockSpec((_D_DOR, _N_EXP), full),
            pl.BlockSpec((_D_VEN, _N_EXP), full),
            pl.BlockSpec((_D_DOR, _N_EXP * _D_EXP), full),
            pl.BlockSpec((_D_VEN, _N_EXP * _D_EXP), full),
            pl.BlockSpec((1, _N_EXP * _D_EXP), full),
            pl.BlockSpec((_N_EXP * _D_EXP, _D_EXP), full),
            pl.BlockSpec((_N_EXP, _D_EXP), full),
            pl.BlockSpec((_D_EXP, _D_OUT), full),
            pl.BlockSpec((1, _D_OUT), full),
            pl.BlockSpec((_D_OUT, _D_DOR), full),
            pl.BlockSpec((_D_OUT, _D_VEN), full),
            pl.BlockSpec((_N_EXP, _N_EXP * _D_EXP), full),
        ],
        out_specs=(
            pl.BlockSpec((block, _D_OUT), tok),
            pl.BlockSpec((block, _D_DOR), tok),
            pl.BlockSpec((block, _D_VEN), tok),
            pl.BlockSpec((block, _N_EXP), tok),
        ),
        out_shape=out_shapes,
        compiler_params=pltpu.CompilerParams(
            dimension_semantics=("arbitrary",),
        ),
    )(dorsal, ventral, gwd, gwv, w1d, w1v, b1row, w2s, b2, woT, borow,
      wfdT, wfvT, expand)


def kernel(dorsal, ventral, gate_w, w1, b1, w2, b2, wo, bo, wfd, wfv):
    return _run(dorsal, ventral, gate_w, w1, b1, w2, b2, wo, bo, wfd, wfv)
